# Initial kernel scaffold; baseline (speedup 1.0000x reference)
import jax
import jax.numpy as jnp
from jax.experimental import pallas as pl


def kernel(x, edge_index, W1, b1, W2, b2, Wc, bc):
    raise NotImplementedError("write your pallas kernel here")


# speedup vs baseline: 25.8904x; 25.8904x over previous
import functools

import jax
import jax.numpy as jnp
from jax import lax
from jax.experimental import pallas as pl
from jax.experimental.pallas import tpu as pltpu
from jax.experimental.pallas import tpu_sc as plsc

NC = 2
NS = 16
NW = NC * NS
CHUNK = 128
NBUF = 4


def _sc_degree(dst2d, *, n_acc, k):
    rpt = n_acc // NS
    zpad = ((rpt + 15) // 16) * 16
    mesh = plsc.VectorSubcoreMesh(core_axis_name="c", subcore_axis_name="s")

    @functools.partial(
        pl.kernel,
        out_type=jax.ShapeDtypeStruct((NC * n_acc,), jnp.float32),
        mesh=mesh,
        compiler_params=pltpu.CompilerParams(use_tc_tiling_on_sc=False),
        scratch_types=[
            pltpu.VMEM((k, CHUNK), jnp.int32),
            pltpu.VMEM((CHUNK,), jnp.float32),
            pltpu.VMEM((zpad,), jnp.float32),
            pltpu.VMEM_SHARED((n_acc,), jnp.float32),
        ],
    )
    def deg_kernel(dst_hbm, out_hbm, idx_v, ones_v, zbuf, acc_sh):
        c = lax.axis_index("c")
        s = lax.axis_index("s")
        w = c * NS + s
        pltpu.sync_copy(dst_hbm.at[pl.ds(pl.multiple_of(w * k, 8), k)], idx_v)
        for i in range(CHUNK // 16):
            ones_v[pl.ds(i * 16, 16)] = jnp.ones((16,), jnp.float32)
        def zfill(i, carry):
            zbuf[pl.ds(i * 16, 16)] = jnp.zeros((16,), jnp.float32)
            return carry
        lax.fori_loop(0, zpad // 16, zfill, 0)
        srow = pl.multiple_of(s * rpt, 8)
        pltpu.sync_copy(zbuf.at[pl.ds(0, rpt)], acc_sh.at[pl.ds(srow, rpt)])
        plsc.subcore_barrier()

        def body(j, carry):
            pltpu.sync_copy(ones_v, acc_sh.at[idx_v.at[j]], add=True)
            return carry

        lax.fori_loop(0, k, body, 0)
        plsc.subcore_barrier()
        orow = pl.multiple_of(c * n_acc + s * rpt, 8)
        pltpu.sync_copy(acc_sh.at[pl.ds(srow, rpt)], zbuf.at[pl.ds(0, rpt)])
        pltpu.sync_copy(zbuf.at[pl.ds(0, rpt)], out_hbm.at[pl.ds(orow, rpt)])

    return deg_kernel(dst2d)


def _sc_gather_scatter(y, src2d, dst2d, *, hh, n_acc, k2):
    rpt = n_acc // NS
    mesh = plsc.VectorSubcoreMesh(core_axis_name="c", subcore_axis_name="s")

    @functools.partial(
        pl.kernel,
        out_type=jax.ShapeDtypeStruct((NC, n_acc, hh), jnp.float32),
        mesh=mesh,
        compiler_params=pltpu.CompilerParams(use_tc_tiling_on_sc=False),
        scratch_types=[
            pltpu.VMEM((k2, CHUNK), jnp.int32),
            pltpu.VMEM((k2, CHUNK), jnp.int32),
            pltpu.VMEM((NBUF, CHUNK, hh), jnp.float32),
            pltpu.VMEM((rpt, hh), jnp.float32),
            pltpu.SemaphoreType.DMA,
            pltpu.SemaphoreType.DMA,
            pltpu.SemaphoreType.DMA,
            pltpu.SemaphoreType.DMA,
            pltpu.VMEM_SHARED((n_acc, hh), jnp.float32),
        ],
    )
    def conv_kernel(y_hbm, src_hbm, dst_hbm, out_hbm,
                    src_v, dst_v, buf_v, zbuf, s0, s1, s2, s3, acc_sh):
        sems = (s0, s1, s2, s3)
        c = lax.axis_index("c")
        s = lax.axis_index("s")
        ytab = y_hbm.at[c]
        wrow = pl.multiple_of(s * k2, 8)
        pltpu.sync_copy(src_hbm.at[pl.ds(wrow, k2)], src_v)
        pltpu.sync_copy(dst_hbm.at[pl.ds(wrow, k2)], dst_v)
        def zfill(i, carry):
            for b2 in range(hh // 16):
                zbuf[i, pl.ds(b2 * 16, 16)] = jnp.zeros((16,), jnp.float32)
            return carry
        lax.fori_loop(0, rpt, zfill, 0)
        srow = pl.multiple_of(s * rpt, 8)
        pltpu.sync_copy(zbuf, acc_sh.at[pl.ds(srow, rpt)])
        plsc.subcore_barrier()

        for b in range(NBUF):
            pltpu.async_copy(ytab.at[src_v.at[b]], buf_v.at[b], sems[b])

        def outer(g, carry):
            for b in range(NBUF):
                j = g * NBUF + b
                pltpu.make_async_copy(
                    ytab.at[src_v.at[j]], buf_v.at[b], sems[b]).wait()
                pltpu.sync_copy(buf_v.at[b], acc_sh.at[dst_v.at[j]], add=True)
                pltpu.async_copy(
                    ytab.at[src_v.at[j + NBUF]], buf_v.at[b], sems[b])
            return carry

        lax.fori_loop(0, k2 // NBUF - 1, outer, 0)
        for b in range(NBUF):
            j = (k2 // NBUF - 1) * NBUF + b
            pltpu.make_async_copy(
                ytab.at[src_v.at[j]], buf_v.at[b], sems[b]).wait()
            pltpu.sync_copy(buf_v.at[b], acc_sh.at[dst_v.at[j]], add=True)

        plsc.subcore_barrier()
        pltpu.sync_copy(acc_sh.at[pl.ds(srow, rpt)], zbuf)
        pltpu.sync_copy(zbuf, out_hbm.at[c, pl.ds(srow, rpt)])

    return conv_kernel(y, src2d, dst2d)


def _tc_y1(x, W1, deg2, *, n_acc):
    n, f = x.shape
    h = W1.shape[1]
    hh = h // NC
    r = n_acc // 16

    def body(x_ref, w_ref, d_ref, y_ref):
        dinv = lax.rsqrt(d_ref[0] + d_ref[1] + 1.0)
        y = dinv * jnp.dot(x_ref[...], w_ref[...],
                           preferred_element_type=jnp.float32)
        y_ref[0] = y[:, :hh]
        y_ref[1] = y[:, hh:]

    return pl.pallas_call(
        body,
        grid=(n_acc // r,),
        in_specs=[
            pl.BlockSpec((r, f), lambda g: (g, 0)),
            pl.BlockSpec((f, h), lambda g: (0, 0)),
            pl.BlockSpec((NC, r, 1), lambda g: (0, g, 0)),
        ],
        out_specs=pl.BlockSpec((NC, r, hh), lambda g: (0, g, 0)),
        out_shape=jax.ShapeDtypeStruct((NC, n_acc, hh), jnp.float32),
    )(x, W1, deg2)


def _tc_mid(acc, y1, deg2, W2, b1, *, n_acc):
    hh_in = y1.shape[2]
    h_out = W2.shape[1]
    hh_out = h_out // NC
    r = n_acc // 16

    def body(a_ref, y_ref, d_ref, w_ref, b_ref, o_ref):
        dinv = lax.rsqrt(d_ref[0] + d_ref[1] + 1.0)
        agg = jnp.concatenate([a_ref[0] + y_ref[0], a_ref[1] + y_ref[1]],
                              axis=1)
        h1 = jnp.maximum(dinv * agg + b_ref[...], 0.0)
        y2 = dinv * jnp.dot(h1, w_ref[...],
                            preferred_element_type=jnp.float32)
        o_ref[0] = y2[:, :hh_out]
        o_ref[1] = y2[:, hh_out:]

    return pl.pallas_call(
        body,
        grid=(n_acc // r,),
        in_specs=[
            pl.BlockSpec((NC, r, hh_in), lambda g: (0, g, 0)),
            pl.BlockSpec((NC, r, hh_in), lambda g: (0, g, 0)),
            pl.BlockSpec((NC, r, 1), lambda g: (0, g, 0)),
            pl.BlockSpec((NC * hh_in, h_out), lambda g: (0, 0)),
            pl.BlockSpec((1, NC * hh_in), lambda g: (0, 0)),
        ],
        out_specs=pl.BlockSpec((NC, r, hh_out), lambda g: (0, g, 0)),
        out_shape=jax.ShapeDtypeStruct((NC, n_acc, hh_out), jnp.float32),
    )(acc, y1, deg2, W2, b1)


def _tc_final(acc, y2, deg2, Wc, b2, bc, *, n, n_acc):
    hh_in = y2.shape[2]
    c_out = Wc.shape[1]
    r = n_acc // 16

    def body(a_ref, y_ref, d_ref, w_ref, b2_ref, bc_ref, o_ref):
        dinv = lax.rsqrt(d_ref[0] + d_ref[1] + 1.0)
        agg = jnp.concatenate([a_ref[0] + y_ref[0], a_ref[1] + y_ref[1]],
                              axis=1)
        h2 = jnp.maximum(dinv * agg + b2_ref[...], 0.0)
        logits = jnp.dot(h2, w_ref[...],
                         preferred_element_type=jnp.float32) + bc_ref[...]
        m = jnp.max(logits, axis=1, keepdims=True)
        lse = jnp.log(jnp.sum(jnp.exp(logits - m), axis=1, keepdims=True)) + m
        o_ref[...] = logits - lse

    return pl.pallas_call(
        body,
        grid=(n_acc // r,),
        in_specs=[
            pl.BlockSpec((NC, r, hh_in), lambda g: (0, g, 0)),
            pl.BlockSpec((NC, r, hh_in), lambda g: (0, g, 0)),
            pl.BlockSpec((NC, r, 1), lambda g: (0, g, 0)),
            pl.BlockSpec((NC * hh_in, c_out), lambda g: (0, 0)),
            pl.BlockSpec((1, NC * hh_in), lambda g: (0, 0)),
            pl.BlockSpec((1, c_out), lambda g: (0, 0)),
        ],
        out_specs=pl.BlockSpec((r, c_out), lambda g: (g, 0)),
        out_shape=jax.ShapeDtypeStruct((n, c_out), jnp.float32),
    )(acc, y2, deg2, Wc, b2, bc)


def kernel(x, edge_index, W1, b1, W2, b2, Wc, bc):
    n, _ = x.shape
    e = edge_index.shape[1]
    h1 = W1.shape[1]
    h2 = W2.shape[1]

    epb = NW * CHUNK * NBUF
    e_pad = ((e + epb - 1) // epb) * epb
    k = e_pad // (NW * CHUNK)
    k2 = e_pad // (NS * CHUNK)
    n_acc = ((n + 1 + 127) // 128) * 128

    padv = jnp.full((e_pad - e,), n, jnp.int32)
    src2d = jnp.concatenate([edge_index[0], padv]).reshape(e_pad // CHUNK, CHUNK)
    dst2d = jnp.concatenate([edge_index[1], padv]).reshape(e_pad // CHUNK, CHUNK)

    deg = _sc_degree(dst2d, n_acc=n_acc, k=k)
    deg2 = deg.reshape(NC, n_acc, 1)

    y1 = _tc_y1(x, W1, deg2, n_acc=n_acc)
    acc1 = _sc_gather_scatter(y1, src2d, dst2d, hh=h1 // NC, n_acc=n_acc, k2=k2)
    y2 = _tc_mid(acc1, y1, deg2, W2, b1.reshape(1, -1), n_acc=n_acc)
    acc2 = _sc_gather_scatter(y2, src2d, dst2d, hh=h2 // NC, n_acc=n_acc, k2=k2)
    return _tc_final(acc2, y2, deg2, Wc, b2.reshape(1, -1), bc.reshape(1, -1),
                     n=n, n_acc=n_acc)


# speedup vs baseline: 26.5861x; 1.0269x over previous
import functools

import jax
import jax.numpy as jnp
from jax import lax
from jax.experimental import pallas as pl
from jax.experimental.pallas import tpu as pltpu
from jax.experimental.pallas import tpu_sc as plsc

NC = 2
NS = 16
NW = NC * NS
CHUNK = 128
NBUF = 4


def _sc_degree(dst2d, *, n_acc, k):
    rpt = n_acc // NS
    zpad = ((rpt + 15) // 16) * 16
    mesh = plsc.VectorSubcoreMesh(core_axis_name="c", subcore_axis_name="s")

    @functools.partial(
        pl.kernel,
        out_type=jax.ShapeDtypeStruct((NC * n_acc,), jnp.float32),
        mesh=mesh,
        compiler_params=pltpu.CompilerParams(use_tc_tiling_on_sc=False),
        scratch_types=[
            pltpu.VMEM((k, CHUNK), jnp.int32),
            pltpu.VMEM((CHUNK,), jnp.float32),
            pltpu.VMEM((zpad,), jnp.float32),
            pltpu.VMEM_SHARED((n_acc,), jnp.float32),
        ],
    )
    def deg_kernel(dst_hbm, out_hbm, idx_v, ones_v, zbuf, acc_sh):
        c = lax.axis_index("c")
        s = lax.axis_index("s")
        w = c * NS + s
        pltpu.sync_copy(dst_hbm.at[pl.ds(pl.multiple_of(w * k, 8), k)], idx_v)
        for i in range(CHUNK // 16):
            ones_v[pl.ds(i * 16, 16)] = jnp.ones((16,), jnp.float32)
        def zfill(i, carry):
            zbuf[pl.ds(i * 16, 16)] = jnp.zeros((16,), jnp.float32)
            return carry
        lax.fori_loop(0, zpad // 16, zfill, 0)
        srow = pl.multiple_of(s * rpt, 8)
        pltpu.sync_copy(zbuf.at[pl.ds(0, rpt)], acc_sh.at[pl.ds(srow, rpt)])
        plsc.subcore_barrier()

        def body(j, carry):
            pltpu.sync_copy(ones_v, acc_sh.at[idx_v.at[j]], add=True)
            return carry

        lax.fori_loop(0, k, body, 0)
        plsc.subcore_barrier()
        orow = pl.multiple_of(c * n_acc + s * rpt, 8)
        pltpu.sync_copy(acc_sh.at[pl.ds(srow, rpt)], zbuf.at[pl.ds(0, rpt)])
        pltpu.sync_copy(zbuf.at[pl.ds(0, rpt)], out_hbm.at[pl.ds(orow, rpt)])

    return deg_kernel(dst2d)


def _sc_gather_scatter(y, src2d, dst2d, *, hh, n_acc, k2):
    rpt = n_acc // NS
    mesh = plsc.VectorSubcoreMesh(core_axis_name="c", subcore_axis_name="s")

    nb = 2 * NBUF
    sem_types = [pltpu.SemaphoreType.DMA] * (2 * nb)

    @functools.partial(
        pl.kernel,
        out_type=jax.ShapeDtypeStruct((NC, n_acc, hh), jnp.float32),
        mesh=mesh,
        compiler_params=pltpu.CompilerParams(use_tc_tiling_on_sc=False),
        scratch_types=[
            pltpu.VMEM((k2, CHUNK), jnp.int32),
            pltpu.VMEM((k2, CHUNK), jnp.int32),
            pltpu.VMEM((nb, CHUNK, hh), jnp.float32),
            pltpu.VMEM((rpt, hh), jnp.float32),
            pltpu.SemaphoreType.DMA,
            pltpu.SemaphoreType.DMA,
            sem_types,
            pltpu.VMEM_SHARED((n_acc, hh), jnp.float32),
        ],
    )
    def conv_kernel(y_hbm, src_hbm, dst_hbm, out_hbm,
                    src_v, dst_v, buf_v, zbuf, isem0, isem1, sems, acc_sh):
        gsem = sems[:nb]
        ssem = sems[nb:]
        c = lax.axis_index("c")
        s = lax.axis_index("s")
        ytab = y_hbm.at[c]
        wrow = pl.multiple_of(s * k2, 8)
        cp_src = pltpu.async_copy(src_hbm.at[pl.ds(wrow, k2)], src_v, isem0)
        cp_dst = pltpu.async_copy(dst_hbm.at[pl.ds(wrow, k2)], dst_v, isem1)
        def zfill(i, carry):
            for b2 in range(hh // 16):
                zbuf[i, pl.ds(b2 * 16, 16)] = jnp.zeros((16,), jnp.float32)
            return carry
        lax.fori_loop(0, rpt, zfill, 0)
        srow = pl.multiple_of(s * rpt, 8)
        pltpu.sync_copy(zbuf, acc_sh.at[pl.ds(srow, rpt)])
        cp_src.wait()
        cp_dst.wait()
        plsc.subcore_barrier()

        def gather(j, b):
            pltpu.async_copy(ytab.at[src_v.at[j]], buf_v.at[b], gsem[b])

        def wait_gather(j, b):
            pltpu.make_async_copy(
                ytab.at[src_v.at[j]], buf_v.at[b], gsem[b]).wait()

        def scatter(j, b):
            pltpu.async_copy(buf_v.at[b], acc_sh.at[dst_v.at[j]], ssem[b],
                             add=True)

        def wait_scatter(j, b):
            pltpu.make_async_copy(
                buf_v.at[b], acc_sh.at[dst_v.at[j]], ssem[b]).wait()

        for j in range(NBUF):
            gather(j, j % nb)
        for j in range(NBUF):
            wait_gather(j, j % nb)
            scatter(j, j % nb)
            gather(j + NBUF, (j + NBUF) % nb)

        def steady(g, carry):
            for b in range(nb):
                j = NBUF + g * nb + b
                bm = (NBUF + b) % nb
                wait_scatter(j - NBUF, b)
                gather(j + NBUF, b)
                wait_gather(j, bm)
                scatter(j, bm)
            return carry

        nsteady = (k2 - 2 * NBUF) // nb
        lax.fori_loop(0, nsteady, steady, 0)
        for jj in range(NBUF):
            j = k2 - NBUF + jj
            wait_gather(j, j % nb)
            scatter(j, j % nb)
        for jj in range(nb):
            j = k2 - nb + jj
            wait_scatter(j, j % nb)

        plsc.subcore_barrier()
        pltpu.sync_copy(acc_sh.at[pl.ds(srow, rpt)], zbuf)
        pltpu.sync_copy(zbuf, out_hbm.at[c, pl.ds(srow, rpt)])

    return conv_kernel(y, src2d, dst2d)


def _tc_y1(x, W1, deg2, *, n_acc):
    n, f = x.shape
    h = W1.shape[1]
    hh = h // NC
    r = n_acc // 16

    def body(x_ref, w_ref, d_ref, y_ref):
        dinv = lax.rsqrt(d_ref[0] + d_ref[1] + 1.0)
        y = dinv * jnp.dot(x_ref[...], w_ref[...],
                           preferred_element_type=jnp.float32)
        y_ref[0] = y[:, :hh]
        y_ref[1] = y[:, hh:]

    return pl.pallas_call(
        body,
        grid=(n_acc // r,),
        in_specs=[
            pl.BlockSpec((r, f), lambda g: (g, 0)),
            pl.BlockSpec((f, h), lambda g: (0, 0)),
            pl.BlockSpec((NC, r, 1), lambda g: (0, g, 0)),
        ],
        out_specs=pl.BlockSpec((NC, r, hh), lambda g: (0, g, 0)),
        out_shape=jax.ShapeDtypeStruct((NC, n_acc, hh), jnp.float32),
    )(x, W1, deg2)


def _tc_mid(acc, y1, deg2, W2, b1, *, n_acc):
    hh_in = y1.shape[2]
    h_out = W2.shape[1]
    hh_out = h_out // NC
    r = n_acc // 16

    def body(a_ref, y_ref, d_ref, w_ref, b_ref, o_ref):
        dinv = lax.rsqrt(d_ref[0] + d_ref[1] + 1.0)
        agg = jnp.concatenate([a_ref[0] + y_ref[0], a_ref[1] + y_ref[1]],
                              axis=1)
        h1 = jnp.maximum(dinv * agg + b_ref[...], 0.0)
        y2 = dinv * jnp.dot(h1, w_ref[...],
                            preferred_element_type=jnp.float32)
        o_ref[0] = y2[:, :hh_out]
        o_ref[1] = y2[:, hh_out:]

    return pl.pallas_call(
        body,
        grid=(n_acc // r,),
        in_specs=[
            pl.BlockSpec((NC, r, hh_in), lambda g: (0, g, 0)),
            pl.BlockSpec((NC, r, hh_in), lambda g: (0, g, 0)),
            pl.BlockSpec((NC, r, 1), lambda g: (0, g, 0)),
            pl.BlockSpec((NC * hh_in, h_out), lambda g: (0, 0)),
            pl.BlockSpec((1, NC * hh_in), lambda g: (0, 0)),
        ],
        out_specs=pl.BlockSpec((NC, r, hh_out), lambda g: (0, g, 0)),
        out_shape=jax.ShapeDtypeStruct((NC, n_acc, hh_out), jnp.float32),
    )(acc, y1, deg2, W2, b1)


def _tc_final(acc, y2, deg2, Wc, b2, bc, *, n, n_acc):
    hh_in = y2.shape[2]
    c_out = Wc.shape[1]
    r = n_acc // 16

    def body(a_ref, y_ref, d_ref, w_ref, b2_ref, bc_ref, o_ref):
        dinv = lax.rsqrt(d_ref[0] + d_ref[1] + 1.0)
        agg = jnp.concatenate([a_ref[0] + y_ref[0], a_ref[1] + y_ref[1]],
                              axis=1)
        h2 = jnp.maximum(dinv * agg + b2_ref[...], 0.0)
        logits = jnp.dot(h2, w_ref[...],
                         preferred_element_type=jnp.float32) + bc_ref[...]
        m = jnp.max(logits, axis=1, keepdims=True)
        lse = jnp.log(jnp.sum(jnp.exp(logits - m), axis=1, keepdims=True)) + m
        o_ref[...] = logits - lse

    return pl.pallas_call(
        body,
        grid=(n_acc // r,),
        in_specs=[
            pl.BlockSpec((NC, r, hh_in), lambda g: (0, g, 0)),
            pl.BlockSpec((NC, r, hh_in), lambda g: (0, g, 0)),
            pl.BlockSpec((NC, r, 1), lambda g: (0, g, 0)),
            pl.BlockSpec((NC * hh_in, c_out), lambda g: (0, 0)),
            pl.BlockSpec((1, NC * hh_in), lambda g: (0, 0)),
            pl.BlockSpec((1, c_out), lambda g: (0, 0)),
        ],
        out_specs=pl.BlockSpec((r, c_out), lambda g: (g, 0)),
        out_shape=jax.ShapeDtypeStruct((n, c_out), jnp.float32),
    )(acc, y2, deg2, Wc, b2, bc)


def kernel(x, edge_index, W1, b1, W2, b2, Wc, bc):
    n, _ = x.shape
    e = edge_index.shape[1]
    h1 = W1.shape[1]
    h2 = W2.shape[1]

    epb = NW * CHUNK * NBUF
    e_pad = ((e + epb - 1) // epb) * epb
    k = e_pad // (NW * CHUNK)
    k2 = e_pad // (NS * CHUNK)
    n_acc = ((n + 1 + 127) // 128) * 128

    padv = jnp.full((e_pad - e,), n, jnp.int32)
    src2d = jnp.concatenate([edge_index[0], padv]).reshape(e_pad // CHUNK, CHUNK)
    dst2d = jnp.concatenate([edge_index[1], padv]).reshape(e_pad // CHUNK, CHUNK)

    deg = _sc_degree(dst2d, n_acc=n_acc, k=k)
    deg2 = deg.reshape(NC, n_acc, 1)

    y1 = _tc_y1(x, W1, deg2, n_acc=n_acc)
    acc1 = _sc_gather_scatter(y1, src2d, dst2d, hh=h1 // NC, n_acc=n_acc, k2=k2)
    y2 = _tc_mid(acc1, y1, deg2, W2, b1.reshape(1, -1), n_acc=n_acc)
    acc2 = _sc_gather_scatter(y2, src2d, dst2d, hh=h2 // NC, n_acc=n_acc, k2=k2)
    return _tc_final(acc2, y2, deg2, Wc, b2.reshape(1, -1), bc.reshape(1, -1),
                     n=n, n_acc=n_acc)


# speedup vs baseline: 26.8492x; 1.0099x over previous
import functools

import jax
import jax.numpy as jnp
from jax import lax
from jax.experimental import pallas as pl
from jax.experimental.pallas import tpu as pltpu
from jax.experimental.pallas import tpu_sc as plsc

NC = 2
NS = 16
NW = NC * NS
CHUNK = 128
NBUF = 8


def _sc_degree(dst2d, *, n_acc, k):
    rpt = n_acc // NS
    zpad = ((rpt + 15) // 16) * 16
    mesh = plsc.VectorSubcoreMesh(core_axis_name="c", subcore_axis_name="s")

    @functools.partial(
        pl.kernel,
        out_type=jax.ShapeDtypeStruct((NC * n_acc,), jnp.float32),
        mesh=mesh,
        compiler_params=pltpu.CompilerParams(use_tc_tiling_on_sc=False),
        scratch_types=[
            pltpu.VMEM((k, CHUNK), jnp.int32),
            pltpu.VMEM((CHUNK,), jnp.float32),
            pltpu.VMEM((zpad,), jnp.float32),
            pltpu.VMEM_SHARED((n_acc,), jnp.float32),
        ],
    )
    def deg_kernel(dst_hbm, out_hbm, idx_v, ones_v, zbuf, acc_sh):
        c = lax.axis_index("c")
        s = lax.axis_index("s")
        w = c * NS + s
        pltpu.sync_copy(dst_hbm.at[pl.ds(pl.multiple_of(w * k, 8), k)], idx_v)
        for i in range(CHUNK // 16):
            ones_v[pl.ds(i * 16, 16)] = jnp.ones((16,), jnp.float32)
        def zfill(i, carry):
            zbuf[pl.ds(i * 16, 16)] = jnp.zeros((16,), jnp.float32)
            return carry
        lax.fori_loop(0, zpad // 16, zfill, 0)
        srow = pl.multiple_of(s * rpt, 8)
        pltpu.sync_copy(zbuf.at[pl.ds(0, rpt)], acc_sh.at[pl.ds(srow, rpt)])
        plsc.subcore_barrier()

        def body(j, carry):
            pltpu.sync_copy(ones_v, acc_sh.at[idx_v.at[j]], add=True)
            return carry

        lax.fori_loop(0, k, body, 0)
        plsc.subcore_barrier()
        orow = pl.multiple_of(c * n_acc + s * rpt, 8)
        pltpu.sync_copy(acc_sh.at[pl.ds(srow, rpt)], zbuf.at[pl.ds(0, rpt)])
        pltpu.sync_copy(zbuf.at[pl.ds(0, rpt)], out_hbm.at[pl.ds(orow, rpt)])

    return deg_kernel(dst2d)


def _sc_gather_scatter(y, src2d, dst2d, *, hh, n_acc, k2, nbuf):
    rpt = n_acc // NS
    mesh = plsc.VectorSubcoreMesh(core_axis_name="c", subcore_axis_name="s")

    sem_types = [pltpu.SemaphoreType.DMA] * nbuf

    @functools.partial(
        pl.kernel,
        out_type=jax.ShapeDtypeStruct((NC, n_acc, hh), jnp.float32),
        mesh=mesh,
        compiler_params=pltpu.CompilerParams(use_tc_tiling_on_sc=False),
        scratch_types=[
            pltpu.VMEM((k2, CHUNK), jnp.int32),
            pltpu.VMEM((k2, CHUNK), jnp.int32),
            pltpu.VMEM((nbuf, CHUNK, hh), jnp.float32),
            pltpu.VMEM((rpt, hh), jnp.float32),
            pltpu.SemaphoreType.DMA,
            pltpu.SemaphoreType.DMA,
            sem_types,
            pltpu.VMEM_SHARED((n_acc, hh), jnp.float32),
        ],
    )
    def conv_kernel(y_hbm, src_hbm, dst_hbm, out_hbm,
                    src_v, dst_v, buf_v, zbuf, isem0, isem1, gsem, acc_sh):
        c = lax.axis_index("c")
        s = lax.axis_index("s")
        ytab = y_hbm.at[c]
        wrow = pl.multiple_of(s * k2, 8)
        cp_src = pltpu.async_copy(src_hbm.at[pl.ds(wrow, k2)], src_v, isem0)
        cp_dst = pltpu.async_copy(dst_hbm.at[pl.ds(wrow, k2)], dst_v, isem1)
        def zfill(i, carry):
            for b2 in range(hh // 16):
                zbuf[i, pl.ds(b2 * 16, 16)] = jnp.zeros((16,), jnp.float32)
            return carry
        lax.fori_loop(0, rpt, zfill, 0)
        srow = pl.multiple_of(s * rpt, 8)
        pltpu.sync_copy(zbuf, acc_sh.at[pl.ds(srow, rpt)])
        cp_src.wait()
        cp_dst.wait()
        plsc.subcore_barrier()

        def gather(j, b):
            pltpu.async_copy(ytab.at[src_v.at[j]], buf_v.at[b], gsem[b])

        def wait_gather(j, b):
            pltpu.make_async_copy(
                ytab.at[src_v.at[j]], buf_v.at[b], gsem[b]).wait()

        def scatter(j, b):
            pltpu.sync_copy(buf_v.at[b], acc_sh.at[dst_v.at[j]], add=True)

        for j in range(nbuf):
            gather(j, j)

        def steady(g, carry):
            for b in range(nbuf):
                j = g * nbuf + b
                wait_gather(j, b)
                scatter(j, b)
                gather(j + nbuf, b)
            return carry

        lax.fori_loop(0, k2 // nbuf - 1, steady, 0)
        for b in range(nbuf):
            j = k2 - nbuf + b
            wait_gather(j, b)
            scatter(j, b)

        plsc.subcore_barrier()
        pltpu.sync_copy(acc_sh.at[pl.ds(srow, rpt)], zbuf)
        pltpu.sync_copy(zbuf, out_hbm.at[c, pl.ds(srow, rpt)])

    return conv_kernel(y, src2d, dst2d)


def _tc_y1(x, W1, deg2, *, n_acc):
    n, f = x.shape
    h = W1.shape[1]
    hh = h // NC
    r = n_acc // 16

    def body(x_ref, w_ref, d_ref, y_ref):
        dinv = lax.rsqrt(d_ref[0] + d_ref[1] + 1.0)
        y = dinv * jnp.dot(x_ref[...], w_ref[...],
                           preferred_element_type=jnp.float32)
        y_ref[0] = y[:, :hh]
        y_ref[1] = y[:, hh:]

    return pl.pallas_call(
        body,
        grid=(n_acc // r,),
        in_specs=[
            pl.BlockSpec((r, f), lambda g: (g, 0)),
            pl.BlockSpec((f, h), lambda g: (0, 0)),
            pl.BlockSpec((NC, r, 1), lambda g: (0, g, 0)),
        ],
        out_specs=pl.BlockSpec((NC, r, hh), lambda g: (0, g, 0)),
        out_shape=jax.ShapeDtypeStruct((NC, n_acc, hh), jnp.float32),
    )(x, W1, deg2)


def _tc_mid(acc, y1, deg2, W2, b1, *, n_acc):
    hh_in = y1.shape[2]
    h_out = W2.shape[1]
    hh_out = h_out // NC
    r = n_acc // 16

    def body(a_ref, y_ref, d_ref, w_ref, b_ref, o_ref):
        dinv = lax.rsqrt(d_ref[0] + d_ref[1] + 1.0)
        agg = jnp.concatenate([a_ref[0] + y_ref[0], a_ref[1] + y_ref[1]],
                              axis=1)
        h1 = jnp.maximum(dinv * agg + b_ref[...], 0.0)
        y2 = dinv * jnp.dot(h1, w_ref[...],
                            preferred_element_type=jnp.float32)
        o_ref[0] = y2[:, :hh_out]
        o_ref[1] = y2[:, hh_out:]

    return pl.pallas_call(
        body,
        grid=(n_acc // r,),
        in_specs=[
            pl.BlockSpec((NC, r, hh_in), lambda g: (0, g, 0)),
            pl.BlockSpec((NC, r, hh_in), lambda g: (0, g, 0)),
            pl.BlockSpec((NC, r, 1), lambda g: (0, g, 0)),
            pl.BlockSpec((NC * hh_in, h_out), lambda g: (0, 0)),
            pl.BlockSpec((1, NC * hh_in), lambda g: (0, 0)),
        ],
        out_specs=pl.BlockSpec((NC, r, hh_out), lambda g: (0, g, 0)),
        out_shape=jax.ShapeDtypeStruct((NC, n_acc, hh_out), jnp.float32),
    )(acc, y1, deg2, W2, b1)


def _tc_final(acc, y2, deg2, Wc, b2, bc, *, n, n_acc):
    hh_in = y2.shape[2]
    c_out = Wc.shape[1]
    r = n_acc // 16

    def body(a_ref, y_ref, d_ref, w_ref, b2_ref, bc_ref, o_ref):
        dinv = lax.rsqrt(d_ref[0] + d_ref[1] + 1.0)
        agg = jnp.concatenate([a_ref[0] + y_ref[0], a_ref[1] + y_ref[1]],
                              axis=1)
        h2 = jnp.maximum(dinv * agg + b2_ref[...], 0.0)
        logits = jnp.dot(h2, w_ref[...],
                         preferred_element_type=jnp.float32) + bc_ref[...]
        m = jnp.max(logits, axis=1, keepdims=True)
        lse = jnp.log(jnp.sum(jnp.exp(logits - m), axis=1, keepdims=True)) + m
        o_ref[...] = logits - lse

    return pl.pallas_call(
        body,
        grid=(n_acc // r,),
        in_specs=[
            pl.BlockSpec((NC, r, hh_in), lambda g: (0, g, 0)),
            pl.BlockSpec((NC, r, hh_in), lambda g: (0, g, 0)),
            pl.BlockSpec((NC, r, 1), lambda g: (0, g, 0)),
            pl.BlockSpec((NC * hh_in, c_out), lambda g: (0, 0)),
            pl.BlockSpec((1, NC * hh_in), lambda g: (0, 0)),
            pl.BlockSpec((1, c_out), lambda g: (0, 0)),
        ],
        out_specs=pl.BlockSpec((r, c_out), lambda g: (g, 0)),
        out_shape=jax.ShapeDtypeStruct((n, c_out), jnp.float32),
    )(acc, y2, deg2, Wc, b2, bc)


def kernel(x, edge_index, W1, b1, W2, b2, Wc, bc):
    n, _ = x.shape
    e = edge_index.shape[1]
    h1 = W1.shape[1]
    h2 = W2.shape[1]

    epb = NW * CHUNK * NBUF
    e_pad = ((e + epb - 1) // epb) * epb
    k = e_pad // (NW * CHUNK)
    k2 = e_pad // (NS * CHUNK)
    n_acc = ((n + 1 + 127) // 128) * 128

    padv = jnp.full((e_pad - e,), n, jnp.int32)
    src2d = jnp.concatenate([edge_index[0], padv]).reshape(e_pad // CHUNK, CHUNK)
    dst2d = jnp.concatenate([edge_index[1], padv]).reshape(e_pad // CHUNK, CHUNK)

    deg = _sc_degree(dst2d, n_acc=n_acc, k=k)
    deg2 = deg.reshape(NC, n_acc, 1)

    y1 = _tc_y1(x, W1, deg2, n_acc=n_acc)
    acc1 = _sc_gather_scatter(y1, src2d, dst2d, hh=h1 // NC, n_acc=n_acc,
                              k2=k2, nbuf=NBUF)
    y2 = _tc_mid(acc1, y1, deg2, W2, b1.reshape(1, -1), n_acc=n_acc)
    acc2 = _sc_gather_scatter(y2, src2d, dst2d, hh=h2 // NC, n_acc=n_acc,
                              k2=k2, nbuf=NBUF)
    return _tc_final(acc2, y2, deg2, Wc, b2.reshape(1, -1), bc.reshape(1, -1),
                     n=n, n_acc=n_acc)


# speedup vs baseline: 37.0603x; 1.3803x over previous
import functools

import jax
import jax.numpy as jnp
from jax import lax
from jax.experimental import pallas as pl
from jax.experimental.pallas import tpu as pltpu
from jax.experimental.pallas import tpu_sc as plsc

NC = 2
NS = 16
NW = NC * NS
CHUNK = 128
NBUF = 8


def _sc_degree(dst2d, *, n_acc, k):
    rpt = n_acc // NS
    zpad = ((rpt + 15) // 16) * 16
    mesh = plsc.VectorSubcoreMesh(core_axis_name="c", subcore_axis_name="s")

    @functools.partial(
        pl.kernel,
        out_type=jax.ShapeDtypeStruct((NC * n_acc,), jnp.float32),
        mesh=mesh,
        compiler_params=pltpu.CompilerParams(use_tc_tiling_on_sc=False),
        scratch_types=[
            pltpu.VMEM((k, CHUNK), jnp.int32),
            pltpu.VMEM((CHUNK,), jnp.float32),
            pltpu.VMEM((zpad,), jnp.float32),
            pltpu.VMEM_SHARED((n_acc,), jnp.float32),
        ],
    )
    def deg_kernel(dst_hbm, out_hbm, idx_v, ones_v, zbuf, acc_sh):
        c = lax.axis_index("c")
        s = lax.axis_index("s")
        w = c * NS + s
        pltpu.sync_copy(dst_hbm.at[pl.ds(pl.multiple_of(w * k, 8), k)], idx_v)
        for i in range(CHUNK // 16):
            ones_v[pl.ds(i * 16, 16)] = jnp.ones((16,), jnp.float32)
        def zfill(i, carry):
            zbuf[pl.ds(i * 16, 16)] = jnp.zeros((16,), jnp.float32)
            return carry
        lax.fori_loop(0, zpad // 16, zfill, 0)
        srow = pl.multiple_of(s * rpt, 8)
        pltpu.sync_copy(zbuf.at[pl.ds(0, rpt)], acc_sh.at[pl.ds(srow, rpt)])
        plsc.subcore_barrier()

        def body(j, carry):
            pltpu.sync_copy(ones_v, acc_sh.at[idx_v.at[j]], add=True)
            return carry

        lax.fori_loop(0, k, body, 0)
        plsc.subcore_barrier()
        orow = pl.multiple_of(c * n_acc + s * rpt, 8)
        pltpu.sync_copy(acc_sh.at[pl.ds(srow, rpt)], zbuf.at[pl.ds(0, rpt)])
        pltpu.sync_copy(zbuf.at[pl.ds(0, rpt)], out_hbm.at[pl.ds(orow, rpt)])

    return deg_kernel(dst2d)


def _sc_gather_scatter(y, src2d, dst2d, *, hh, n_acc, k2, nbuf):
    rpt = n_acc // NS
    mesh = plsc.VectorSubcoreMesh(core_axis_name="c", subcore_axis_name="s")

    sem_types = [pltpu.SemaphoreType.DMA] * nbuf
    nfull = rpt // CHUNK
    rem = rpt % CHUNK

    @functools.partial(
        pl.kernel,
        out_type=jax.ShapeDtypeStruct((NC, n_acc, hh), jnp.float32),
        mesh=mesh,
        compiler_params=pltpu.CompilerParams(use_tc_tiling_on_sc=False),
        scratch_types=[
            pltpu.VMEM((k2, CHUNK), jnp.int32),
            pltpu.VMEM((k2, CHUNK), jnp.int32),
            pltpu.VMEM((nbuf, CHUNK, hh), jnp.float32),
            pltpu.SemaphoreType.DMA,
            pltpu.SemaphoreType.DMA,
            sem_types,
            pltpu.VMEM_SHARED((n_acc, hh), jnp.float32),
            pltpu.VMEM_SHARED((n_acc, hh), jnp.float32),
        ],
    )
    def conv_kernel(y_hbm, src_hbm, dst_hbm, out_hbm,
                    src_v, dst_v, buf_v, isem0, isem1, gsem, y_sh, acc_sh):
        c = lax.axis_index("c")
        s = lax.axis_index("s")
        wrow = pl.multiple_of(s * k2, 8)
        cp_src = pltpu.async_copy(src_hbm.at[pl.ds(wrow, k2)], src_v, isem0)
        cp_dst = pltpu.async_copy(dst_hbm.at[pl.ds(wrow, k2)], dst_v, isem1)

        srow = pl.multiple_of(s * rpt, 8)
        for i in range(nfull):
            r0 = pl.multiple_of(srow + i * CHUNK, 8)
            pltpu.sync_copy(y_hbm.at[c, pl.ds(r0, CHUNK)], buf_v.at[0])
            pltpu.sync_copy(buf_v.at[0], y_sh.at[pl.ds(r0, CHUNK)])
        if rem:
            r0 = pl.multiple_of(srow + nfull * CHUNK, 8)
            pltpu.sync_copy(y_hbm.at[c, pl.ds(r0, rem)],
                            buf_v.at[0, pl.ds(0, rem)])
            pltpu.sync_copy(buf_v.at[0, pl.ds(0, rem)],
                            y_sh.at[pl.ds(r0, rem)])

        def zrow(i, carry):
            for b2 in range(hh // 16):
                buf_v[0, i, pl.ds(b2 * 16, 16)] = jnp.zeros((16,), jnp.float32)
            return carry
        lax.fori_loop(0, CHUNK, zrow, 0)
        for i in range(nfull):
            r0 = pl.multiple_of(srow + i * CHUNK, 8)
            pltpu.sync_copy(buf_v.at[0], acc_sh.at[pl.ds(r0, CHUNK)])
        if rem:
            r0 = pl.multiple_of(srow + nfull * CHUNK, 8)
            pltpu.sync_copy(buf_v.at[0, pl.ds(0, rem)],
                            acc_sh.at[pl.ds(r0, rem)])
        cp_src.wait()
        cp_dst.wait()
        plsc.subcore_barrier()

        def gather(j, b):
            pltpu.async_copy(y_sh.at[src_v.at[j]], buf_v.at[b], gsem[b])

        def wait_gather(j, b):
            pltpu.make_async_copy(
                y_sh.at[src_v.at[j]], buf_v.at[b], gsem[b]).wait()

        def scatter(j, b):
            pltpu.sync_copy(buf_v.at[b], acc_sh.at[dst_v.at[j]], add=True)

        for j in range(nbuf):
            gather(j, j)

        def steady(g, carry):
            for b in range(nbuf):
                j = g * nbuf + b
                wait_gather(j, b)
                scatter(j, b)
                gather(j + nbuf, b)
            return carry

        lax.fori_loop(0, k2 // nbuf - 1, steady, 0)
        for b in range(nbuf):
            j = k2 - nbuf + b
            wait_gather(j, b)
            scatter(j, b)

        plsc.subcore_barrier()
        for i in range(nfull):
            r0 = pl.multiple_of(srow + i * CHUNK, 8)
            pltpu.sync_copy(acc_sh.at[pl.ds(r0, CHUNK)], buf_v.at[0])
            pltpu.sync_copy(buf_v.at[0], out_hbm.at[c, pl.ds(r0, CHUNK)])
        if rem:
            r0 = pl.multiple_of(srow + nfull * CHUNK, 8)
            pltpu.sync_copy(acc_sh.at[pl.ds(r0, rem)],
                            buf_v.at[0, pl.ds(0, rem)])
            pltpu.sync_copy(buf_v.at[0, pl.ds(0, rem)],
                            out_hbm.at[c, pl.ds(r0, rem)])

    return conv_kernel(y, src2d, dst2d)


def _tc_y1(x, W1, deg2, *, n_acc):
    n, f = x.shape
    h = W1.shape[1]
    hh = h // NC
    r = n_acc // 16

    def body(x_ref, w_ref, d_ref, y_ref):
        dinv = lax.rsqrt(d_ref[0] + d_ref[1] + 1.0)
        y = dinv * jnp.dot(x_ref[...], w_ref[...],
                           preferred_element_type=jnp.float32)
        y_ref[0] = y[:, :hh]
        y_ref[1] = y[:, hh:]

    return pl.pallas_call(
        body,
        grid=(n_acc // r,),
        in_specs=[
            pl.BlockSpec((r, f), lambda g: (g, 0)),
            pl.BlockSpec((f, h), lambda g: (0, 0)),
            pl.BlockSpec((NC, r, 1), lambda g: (0, g, 0)),
        ],
        out_specs=pl.BlockSpec((NC, r, hh), lambda g: (0, g, 0)),
        out_shape=jax.ShapeDtypeStruct((NC, n_acc, hh), jnp.float32),
    )(x, W1, deg2)


def _tc_mid(acc, y1, deg2, W2, b1, *, n_acc):
    hh_in = y1.shape[2]
    h_out = W2.shape[1]
    hh_out = h_out // NC
    r = n_acc // 16

    def body(a_ref, y_ref, d_ref, w_ref, b_ref, o_ref):
        dinv = lax.rsqrt(d_ref[0] + d_ref[1] + 1.0)
        agg = jnp.concatenate([a_ref[0] + y_ref[0], a_ref[1] + y_ref[1]],
                              axis=1)
        h1 = jnp.maximum(dinv * agg + b_ref[...], 0.0)
        y2 = dinv * jnp.dot(h1, w_ref[...],
                            preferred_element_type=jnp.float32)
        o_ref[0] = y2[:, :hh_out]
        o_ref[1] = y2[:, hh_out:]

    return pl.pallas_call(
        body,
        grid=(n_acc // r,),
        in_specs=[
            pl.BlockSpec((NC, r, hh_in), lambda g: (0, g, 0)),
            pl.BlockSpec((NC, r, hh_in), lambda g: (0, g, 0)),
            pl.BlockSpec((NC, r, 1), lambda g: (0, g, 0)),
            pl.BlockSpec((NC * hh_in, h_out), lambda g: (0, 0)),
            pl.BlockSpec((1, NC * hh_in), lambda g: (0, 0)),
        ],
        out_specs=pl.BlockSpec((NC, r, hh_out), lambda g: (0, g, 0)),
        out_shape=jax.ShapeDtypeStruct((NC, n_acc, hh_out), jnp.float32),
    )(acc, y1, deg2, W2, b1)


def _tc_final(acc, y2, deg2, Wc, b2, bc, *, n, n_acc):
    hh_in = y2.shape[2]
    c_out = Wc.shape[1]
    r = n_acc // 16

    def body(a_ref, y_ref, d_ref, w_ref, b2_ref, bc_ref, o_ref):
        dinv = lax.rsqrt(d_ref[0] + d_ref[1] + 1.0)
        agg = jnp.concatenate([a_ref[0] + y_ref[0], a_ref[1] + y_ref[1]],
                              axis=1)
        h2 = jnp.maximum(dinv * agg + b2_ref[...], 0.0)
        logits = jnp.dot(h2, w_ref[...],
                         preferred_element_type=jnp.float32) + bc_ref[...]
        m = jnp.max(logits, axis=1, keepdims=True)
        lse = jnp.log(jnp.sum(jnp.exp(logits - m), axis=1, keepdims=True)) + m
        o_ref[...] = logits - lse

    return pl.pallas_call(
        body,
        grid=(n_acc // r,),
        in_specs=[
            pl.BlockSpec((NC, r, hh_in), lambda g: (0, g, 0)),
            pl.BlockSpec((NC, r, hh_in), lambda g: (0, g, 0)),
            pl.BlockSpec((NC, r, 1), lambda g: (0, g, 0)),
            pl.BlockSpec((NC * hh_in, c_out), lambda g: (0, 0)),
            pl.BlockSpec((1, NC * hh_in), lambda g: (0, 0)),
            pl.BlockSpec((1, c_out), lambda g: (0, 0)),
        ],
        out_specs=pl.BlockSpec((r, c_out), lambda g: (g, 0)),
        out_shape=jax.ShapeDtypeStruct((n, c_out), jnp.float32),
    )(acc, y2, deg2, Wc, b2, bc)


def kernel(x, edge_index, W1, b1, W2, b2, Wc, bc):
    n, _ = x.shape
    e = edge_index.shape[1]
    h1 = W1.shape[1]
    h2 = W2.shape[1]

    epb = NW * CHUNK * NBUF
    e_pad = ((e + epb - 1) // epb) * epb
    k = e_pad // (NW * CHUNK)
    k2 = e_pad // (NS * CHUNK)
    n_acc = ((n + 1 + 127) // 128) * 128

    padv = jnp.full((e_pad - e,), n, jnp.int32)
    src2d = jnp.concatenate([edge_index[0], padv]).reshape(e_pad // CHUNK, CHUNK)
    dst2d = jnp.concatenate([edge_index[1], padv]).reshape(e_pad // CHUNK, CHUNK)

    deg = _sc_degree(dst2d, n_acc=n_acc, k=k)
    deg2 = deg.reshape(NC, n_acc, 1)

    y1 = _tc_y1(x, W1, deg2, n_acc=n_acc)
    acc1 = _sc_gather_scatter(y1, src2d, dst2d, hh=h1 // NC, n_acc=n_acc,
                              k2=k2, nbuf=NBUF)
    y2 = _tc_mid(acc1, y1, deg2, W2, b1.reshape(1, -1), n_acc=n_acc)
    acc2 = _sc_gather_scatter(y2, src2d, dst2d, hh=h2 // NC, n_acc=n_acc,
                              k2=k2, nbuf=NBUF)
    return _tc_final(acc2, y2, deg2, Wc, b2.reshape(1, -1), bc.reshape(1, -1),
                     n=n, n_acc=n_acc)


# speedup vs baseline: 38.0464x; 1.0266x over previous
import functools

import jax
import jax.numpy as jnp
from jax import lax
from jax.experimental import pallas as pl
from jax.experimental.pallas import tpu as pltpu
from jax.experimental.pallas import tpu_sc as plsc

NC = 2
NS = 16
NW = NC * NS
CHUNK = 128
NBUF = 8


def _sc_degree(dst2d, *, n_acc, k):
    rpt = n_acc // NS
    zpad = ((rpt + 15) // 16) * 16
    mesh = plsc.VectorSubcoreMesh(core_axis_name="c", subcore_axis_name="s")

    @functools.partial(
        pl.kernel,
        out_type=jax.ShapeDtypeStruct((NC * n_acc,), jnp.float32),
        mesh=mesh,
        compiler_params=pltpu.CompilerParams(use_tc_tiling_on_sc=False),
        scratch_types=[
            pltpu.VMEM((k, CHUNK), jnp.int32),
            pltpu.VMEM((CHUNK,), jnp.float32),
            pltpu.VMEM((zpad,), jnp.float32),
            pltpu.VMEM_SHARED((n_acc,), jnp.float32),
        ],
    )
    def deg_kernel(dst_hbm, out_hbm, idx_v, ones_v, zbuf, acc_sh):
        c = lax.axis_index("c")
        s = lax.axis_index("s")
        w = c * NS + s
        pltpu.sync_copy(dst_hbm.at[pl.ds(pl.multiple_of(w * k, 8), k)], idx_v)
        for i in range(CHUNK // 16):
            ones_v[pl.ds(i * 16, 16)] = jnp.ones((16,), jnp.float32)
        def zfill(i, carry):
            zbuf[pl.ds(i * 16, 16)] = jnp.zeros((16,), jnp.float32)
            return carry
        lax.fori_loop(0, zpad // 16, zfill, 0)
        srow = pl.multiple_of(s * rpt, 8)
        pltpu.sync_copy(zbuf.at[pl.ds(0, rpt)], acc_sh.at[pl.ds(srow, rpt)])
        plsc.subcore_barrier()

        def body(j, carry):
            pltpu.sync_copy(ones_v, acc_sh.at[idx_v.at[j]], add=True)
            return carry

        lax.fori_loop(0, k, body, 0)
        plsc.subcore_barrier()
        orow = pl.multiple_of(c * n_acc + s * rpt, 8)
        pltpu.sync_copy(acc_sh.at[pl.ds(srow, rpt)], zbuf.at[pl.ds(0, rpt)])
        pltpu.sync_copy(zbuf.at[pl.ds(0, rpt)], out_hbm.at[pl.ds(orow, rpt)])

    return deg_kernel(dst2d)


def _sc_gather_scatter(y, src2d, dst2d, *, hh, n_acc, k2, nbuf):
    rpt = n_acc // NS
    mesh = plsc.VectorSubcoreMesh(core_axis_name="c", subcore_axis_name="s")

    sem_types = [pltpu.SemaphoreType.DMA] * nbuf
    nfull = rpt // CHUNK
    rem = rpt % CHUNK

    @functools.partial(
        pl.kernel,
        out_type=jax.ShapeDtypeStruct((NC, n_acc, hh), jnp.float32),
        mesh=mesh,
        compiler_params=pltpu.CompilerParams(use_tc_tiling_on_sc=False),
        scratch_types=[
            pltpu.VMEM((k2, CHUNK), jnp.int32),
            pltpu.VMEM((k2, CHUNK), jnp.int32),
            pltpu.VMEM((nbuf, CHUNK, hh), jnp.float32),
            pltpu.SemaphoreType.DMA,
            pltpu.SemaphoreType.DMA,
            pltpu.SemaphoreType.DMA,
            sem_types,
            pltpu.VMEM_SHARED((n_acc, hh), jnp.float32),
            pltpu.VMEM_SHARED((n_acc, hh), jnp.float32),
        ],
    )
    def conv_kernel(y_hbm, src_hbm, dst_hbm, out_hbm,
                    src_v, dst_v, buf_v, isem0, isem1, ssem, gsem,
                    y_sh, acc_sh):
        c = lax.axis_index("c")
        s = lax.axis_index("s")
        wrow = pl.multiple_of(s * k2, 8)
        cp_src = pltpu.async_copy(src_hbm.at[pl.ds(wrow, k2)], src_v, isem0)
        cp_dst = pltpu.async_copy(dst_hbm.at[pl.ds(wrow, k2)], dst_v, isem1)

        srow = pl.multiple_of(s * rpt, 8)
        for i in range(nfull):
            r0 = pl.multiple_of(srow + i * CHUNK, 8)
            pltpu.sync_copy(y_hbm.at[c, pl.ds(r0, CHUNK)], buf_v.at[0])
            pltpu.sync_copy(buf_v.at[0], y_sh.at[pl.ds(r0, CHUNK)])
        if rem:
            r0 = pl.multiple_of(srow + nfull * CHUNK, 8)
            pltpu.sync_copy(y_hbm.at[c, pl.ds(r0, rem)],
                            buf_v.at[0, pl.ds(0, rem)])
            pltpu.sync_copy(buf_v.at[0, pl.ds(0, rem)],
                            y_sh.at[pl.ds(r0, rem)])

        def zrow(i, carry):
            for b2 in range(hh // 16):
                buf_v[0, i, pl.ds(b2 * 16, 16)] = jnp.zeros((16,), jnp.float32)
            return carry
        lax.fori_loop(0, CHUNK, zrow, 0)
        for i in range(nfull):
            r0 = pl.multiple_of(srow + i * CHUNK, 8)
            pltpu.sync_copy(buf_v.at[0], acc_sh.at[pl.ds(r0, CHUNK)])
        if rem:
            r0 = pl.multiple_of(srow + nfull * CHUNK, 8)
            pltpu.sync_copy(buf_v.at[0, pl.ds(0, rem)],
                            acc_sh.at[pl.ds(r0, rem)])
        cp_src.wait()
        cp_dst.wait()
        plsc.subcore_barrier()

        def gather(j, b):
            pltpu.async_copy(y_sh.at[src_v.at[j]], buf_v.at[b], gsem[b])

        def wait_gather(j, b):
            pltpu.make_async_copy(
                y_sh.at[src_v.at[j]], buf_v.at[b], gsem[b]).wait()

        def scatter(j, b):
            pltpu.async_copy(buf_v.at[b], acc_sh.at[dst_v.at[j]], ssem,
                             add=True)

        def drain_scatter(b):
            pltpu.make_async_copy(y_hbm.at[c, pl.ds(0, CHUNK)], buf_v.at[b],
                                  ssem).wait()

        for b in range(nbuf):
            gather(b, b)
        for b in range(nbuf):
            wait_gather(b, b)
            scatter(b, b)

        def steady(g, carry):
            for b in range(nbuf):
                drain_scatter(b)
                gather(g * nbuf + b, b)
            for b in range(nbuf):
                wait_gather(g * nbuf + b, b)
                scatter(g * nbuf + b, b)
            return carry

        lax.fori_loop(1, k2 // nbuf, steady, 0)
        for b in range(nbuf):
            drain_scatter(b)

        plsc.subcore_barrier()
        for i in range(nfull):
            r0 = pl.multiple_of(srow + i * CHUNK, 8)
            pltpu.sync_copy(acc_sh.at[pl.ds(r0, CHUNK)], buf_v.at[0])
            pltpu.sync_copy(buf_v.at[0], out_hbm.at[c, pl.ds(r0, CHUNK)])
        if rem:
            r0 = pl.multiple_of(srow + nfull * CHUNK, 8)
            pltpu.sync_copy(acc_sh.at[pl.ds(r0, rem)],
                            buf_v.at[0, pl.ds(0, rem)])
            pltpu.sync_copy(buf_v.at[0, pl.ds(0, rem)],
                            out_hbm.at[c, pl.ds(r0, rem)])

    return conv_kernel(y, src2d, dst2d)


def _tc_y1(x, W1, deg2, *, n_acc):
    n, f = x.shape
    h = W1.shape[1]
    hh = h // NC
    r = n_acc // 16

    def body(x_ref, w_ref, d_ref, y_ref):
        dinv = lax.rsqrt(d_ref[0] + d_ref[1] + 1.0)
        y = dinv * jnp.dot(x_ref[...], w_ref[...],
                           preferred_element_type=jnp.float32)
        y_ref[0] = y[:, :hh]
        y_ref[1] = y[:, hh:]

    return pl.pallas_call(
        body,
        grid=(n_acc // r,),
        in_specs=[
            pl.BlockSpec((r, f), lambda g: (g, 0)),
            pl.BlockSpec((f, h), lambda g: (0, 0)),
            pl.BlockSpec((NC, r, 1), lambda g: (0, g, 0)),
        ],
        out_specs=pl.BlockSpec((NC, r, hh), lambda g: (0, g, 0)),
        out_shape=jax.ShapeDtypeStruct((NC, n_acc, hh), jnp.float32),
    )(x, W1, deg2)


def _tc_mid(acc, y1, deg2, W2, b1, *, n_acc):
    hh_in = y1.shape[2]
    h_out = W2.shape[1]
    hh_out = h_out // NC
    r = n_acc // 16

    def body(a_ref, y_ref, d_ref, w_ref, b_ref, o_ref):
        dinv = lax.rsqrt(d_ref[0] + d_ref[1] + 1.0)
        agg = jnp.concatenate([a_ref[0] + y_ref[0], a_ref[1] + y_ref[1]],
                              axis=1)
        h1 = jnp.maximum(dinv * agg + b_ref[...], 0.0)
        y2 = dinv * jnp.dot(h1, w_ref[...],
                            preferred_element_type=jnp.float32)
        o_ref[0] = y2[:, :hh_out]
        o_ref[1] = y2[:, hh_out:]

    return pl.pallas_call(
        body,
        grid=(n_acc // r,),
        in_specs=[
            pl.BlockSpec((NC, r, hh_in), lambda g: (0, g, 0)),
            pl.BlockSpec((NC, r, hh_in), lambda g: (0, g, 0)),
            pl.BlockSpec((NC, r, 1), lambda g: (0, g, 0)),
            pl.BlockSpec((NC * hh_in, h_out), lambda g: (0, 0)),
            pl.BlockSpec((1, NC * hh_in), lambda g: (0, 0)),
        ],
        out_specs=pl.BlockSpec((NC, r, hh_out), lambda g: (0, g, 0)),
        out_shape=jax.ShapeDtypeStruct((NC, n_acc, hh_out), jnp.float32),
    )(acc, y1, deg2, W2, b1)


def _tc_final(acc, y2, deg2, Wc, b2, bc, *, n, n_acc):
    hh_in = y2.shape[2]
    c_out = Wc.shape[1]
    r = n_acc // 16

    def body(a_ref, y_ref, d_ref, w_ref, b2_ref, bc_ref, o_ref):
        dinv = lax.rsqrt(d_ref[0] + d_ref[1] + 1.0)
        agg = jnp.concatenate([a_ref[0] + y_ref[0], a_ref[1] + y_ref[1]],
                              axis=1)
        h2 = jnp.maximum(dinv * agg + b2_ref[...], 0.0)
        logits = jnp.dot(h2, w_ref[...],
                         preferred_element_type=jnp.float32) + bc_ref[...]
        m = jnp.max(logits, axis=1, keepdims=True)
        lse = jnp.log(jnp.sum(jnp.exp(logits - m), axis=1, keepdims=True)) + m
        o_ref[...] = logits - lse

    return pl.pallas_call(
        body,
        grid=(n_acc // r,),
        in_specs=[
            pl.BlockSpec((NC, r, hh_in), lambda g: (0, g, 0)),
            pl.BlockSpec((NC, r, hh_in), lambda g: (0, g, 0)),
            pl.BlockSpec((NC, r, 1), lambda g: (0, g, 0)),
            pl.BlockSpec((NC * hh_in, c_out), lambda g: (0, 0)),
            pl.BlockSpec((1, NC * hh_in), lambda g: (0, 0)),
            pl.BlockSpec((1, c_out), lambda g: (0, 0)),
        ],
        out_specs=pl.BlockSpec((r, c_out), lambda g: (g, 0)),
        out_shape=jax.ShapeDtypeStruct((n, c_out), jnp.float32),
    )(acc, y2, deg2, Wc, b2, bc)


def kernel(x, edge_index, W1, b1, W2, b2, Wc, bc):
    n, _ = x.shape
    e = edge_index.shape[1]
    h1 = W1.shape[1]
    h2 = W2.shape[1]

    epb = NW * CHUNK * NBUF
    e_pad = ((e + epb - 1) // epb) * epb
    k = e_pad // (NW * CHUNK)
    k2 = e_pad // (NS * CHUNK)
    n_acc = ((n + 1 + 127) // 128) * 128

    padv = jnp.full((e_pad - e,), n, jnp.int32)
    src2d = jnp.concatenate([edge_index[0], padv]).reshape(e_pad // CHUNK, CHUNK)
    dst2d = jnp.concatenate([edge_index[1], padv]).reshape(e_pad // CHUNK, CHUNK)

    deg = _sc_degree(dst2d, n_acc=n_acc, k=k)
    deg2 = deg.reshape(NC, n_acc, 1)

    y1 = _tc_y1(x, W1, deg2, n_acc=n_acc)
    acc1 = _sc_gather_scatter(y1, src2d, dst2d, hh=h1 // NC, n_acc=n_acc,
                              k2=k2, nbuf=NBUF)
    y2 = _tc_mid(acc1, y1, deg2, W2, b1.reshape(1, -1), n_acc=n_acc)
    acc2 = _sc_gather_scatter(y2, src2d, dst2d, hh=h2 // NC, n_acc=n_acc,
                              k2=k2, nbuf=NBUF)
    return _tc_final(acc2, y2, deg2, Wc, b2.reshape(1, -1), bc.reshape(1, -1),
                     n=n, n_acc=n_acc)


# speedup vs baseline: 39.2841x; 1.0325x over previous
import functools

import jax
import jax.numpy as jnp
from jax import lax
from jax.experimental import pallas as pl
from jax.experimental.pallas import tpu as pltpu
from jax.experimental.pallas import tpu_sc as plsc

NC = 2
NS = 16
NW = NC * NS
CHUNK = 128
NBUF = 8


def _sc_degree(dst2d, *, n_acc, k):
    rpt = n_acc // NS
    zpad = ((rpt + 15) // 16) * 16
    mesh = plsc.VectorSubcoreMesh(core_axis_name="c", subcore_axis_name="s")

    @functools.partial(
        pl.kernel,
        out_type=jax.ShapeDtypeStruct((NC * n_acc,), jnp.float32),
        mesh=mesh,
        compiler_params=pltpu.CompilerParams(use_tc_tiling_on_sc=False),
        scratch_types=[
            pltpu.VMEM((k, CHUNK), jnp.int32),
            pltpu.VMEM((CHUNK,), jnp.float32),
            pltpu.VMEM((zpad,), jnp.float32),
            pltpu.VMEM_SHARED((n_acc,), jnp.float32),
        ],
    )
    def deg_kernel(dst_hbm, out_hbm, idx_v, ones_v, zbuf, acc_sh):
        c = lax.axis_index("c")
        s = lax.axis_index("s")
        w = c * NS + s
        pltpu.sync_copy(dst_hbm.at[pl.ds(pl.multiple_of(w * k, 8), k)], idx_v)
        for i in range(CHUNK // 16):
            ones_v[pl.ds(i * 16, 16)] = jnp.ones((16,), jnp.float32)
        def zfill(i, carry):
            zbuf[pl.ds(i * 16, 16)] = jnp.zeros((16,), jnp.float32)
            return carry
        lax.fori_loop(0, zpad // 16, zfill, 0)
        srow = pl.multiple_of(s * rpt, 8)
        pltpu.sync_copy(zbuf.at[pl.ds(0, rpt)], acc_sh.at[pl.ds(srow, rpt)])
        plsc.subcore_barrier()

        def body(j, carry):
            pltpu.sync_copy(ones_v, acc_sh.at[idx_v.at[j]], add=True)
            return carry

        lax.fori_loop(0, k, body, 0)
        plsc.subcore_barrier()
        orow = pl.multiple_of(c * n_acc + s * rpt, 8)
        pltpu.sync_copy(acc_sh.at[pl.ds(srow, rpt)], zbuf.at[pl.ds(0, rpt)])
        pltpu.sync_copy(zbuf.at[pl.ds(0, rpt)], out_hbm.at[pl.ds(orow, rpt)])

    return deg_kernel(dst2d)


def _sc_gather_scatter(y, src2d, dst2d, *, hh, n_acc, k2, nbuf):
    rpt = n_acc // NS
    mesh = plsc.VectorSubcoreMesh(core_axis_name="c", subcore_axis_name="s")

    sem_types = [pltpu.SemaphoreType.DMA] * nbuf
    nfull = rpt // CHUNK
    rem = rpt % CHUNK

    @functools.partial(
        pl.kernel,
        out_type=jax.ShapeDtypeStruct((NC, n_acc, hh), jnp.float32),
        mesh=mesh,
        compiler_params=pltpu.CompilerParams(use_tc_tiling_on_sc=False),
        scratch_types=[
            pltpu.VMEM((k2, CHUNK), jnp.int32),
            pltpu.VMEM((k2, CHUNK), jnp.int32),
            pltpu.VMEM((nbuf, CHUNK, hh), jnp.float32),
            pltpu.SemaphoreType.DMA,
            pltpu.SemaphoreType.DMA,
            pltpu.SemaphoreType.DMA,
            sem_types,
            pltpu.VMEM_SHARED((n_acc, hh), jnp.float32),
            pltpu.VMEM_SHARED((n_acc, hh), jnp.float32),
        ],
    )
    def conv_kernel(y_hbm, src_hbm, dst_hbm, out_hbm,
                    src_v, dst_v, buf_v, isem0, isem1, ssem, gsem,
                    y_sh, acc_sh):
        c = lax.axis_index("c")
        s = lax.axis_index("s")
        wrow = pl.multiple_of(s * k2, 8)
        cp_src = pltpu.async_copy(src_hbm.at[pl.ds(wrow, k2)], src_v, isem0)
        cp_dst = pltpu.async_copy(dst_hbm.at[pl.ds(wrow, k2)], dst_v, isem1)

        srow = pl.multiple_of(s * rpt, 8)
        npc = nfull + (1 if rem else 0)
        stage = []
        for i in range(nfull):
            r0 = pl.multiple_of(srow + i * CHUNK, 8)
            stage.append((r0, CHUNK, pltpu.async_copy(
                y_hbm.at[c, pl.ds(r0, CHUNK)], buf_v.at[i], gsem[i])))
        if rem:
            r0 = pl.multiple_of(srow + nfull * CHUNK, 8)
            stage.append((r0, rem, pltpu.async_copy(
                y_hbm.at[c, pl.ds(r0, rem)],
                buf_v.at[nfull, pl.ds(0, rem)], gsem[nfull])))

        zb = nbuf - 1
        def zrow(i, carry):
            for b2 in range(hh // 16):
                buf_v[zb, i, pl.ds(b2 * 16, 16)] = jnp.zeros((16,),
                                                             jnp.float32)
            return carry
        lax.fori_loop(0, CHUNK, zrow, 0)
        zsplat = []
        for i, (r0, cnt, _) in enumerate(stage):
            zsplat.append(pltpu.async_copy(
                buf_v.at[zb, pl.ds(0, cnt)], acc_sh.at[pl.ds(r0, cnt)],
                ssem))
        for i, (r0, cnt, cp) in enumerate(stage):
            cp.wait()
            pltpu.sync_copy(buf_v.at[i, pl.ds(0, cnt)],
                            y_sh.at[pl.ds(r0, cnt)])
        for cp in zsplat:
            cp.wait()
        cp_src.wait()
        cp_dst.wait()
        plsc.subcore_barrier()

        def gather(j, b):
            pltpu.async_copy(y_sh.at[src_v.at[j]], buf_v.at[b], gsem[b])

        def wait_gather(j, b):
            pltpu.make_async_copy(
                y_sh.at[src_v.at[j]], buf_v.at[b], gsem[b]).wait()

        def scatter(j, b):
            pltpu.async_copy(buf_v.at[b], acc_sh.at[dst_v.at[j]], ssem,
                             add=True)

        def drain_scatter(b):
            pltpu.make_async_copy(y_hbm.at[c, pl.ds(0, CHUNK)], buf_v.at[b],
                                  ssem).wait()

        for b in range(nbuf):
            gather(b, b)
        for b in range(nbuf):
            wait_gather(b, b)
            scatter(b, b)

        def steady(g, carry):
            for b in range(nbuf):
                drain_scatter(b)
                gather(g * nbuf + b, b)
            for b in range(nbuf):
                wait_gather(g * nbuf + b, b)
                scatter(g * nbuf + b, b)
            return carry

        lax.fori_loop(1, k2 // nbuf, steady, 0)
        for b in range(nbuf):
            drain_scatter(b)

        plsc.subcore_barrier()
        outcp = []
        for i, (r0, cnt, _) in enumerate(stage):
            outcp.append(pltpu.async_copy(
                acc_sh.at[pl.ds(r0, cnt)], buf_v.at[i, pl.ds(0, cnt)],
                gsem[i]))
        final = []
        for i, (r0, cnt, _) in enumerate(stage):
            outcp[i].wait()
            final.append(pltpu.async_copy(
                buf_v.at[i, pl.ds(0, cnt)], out_hbm.at[c, pl.ds(r0, cnt)],
                ssem))
        for cp in final:
            cp.wait()

    return conv_kernel(y, src2d, dst2d)


def _tc_y1(x, W1, deg2, *, n_acc):
    n, f = x.shape
    h = W1.shape[1]
    hh = h // NC
    r = n_acc // 16

    def body(x_ref, w_ref, d_ref, y_ref):
        dinv = lax.rsqrt(d_ref[0] + d_ref[1] + 1.0)
        y = dinv * jnp.dot(x_ref[...], w_ref[...],
                           preferred_element_type=jnp.float32)
        y_ref[0] = y[:, :hh]
        y_ref[1] = y[:, hh:]

    return pl.pallas_call(
        body,
        grid=(n_acc // r,),
        in_specs=[
            pl.BlockSpec((r, f), lambda g: (g, 0)),
            pl.BlockSpec((f, h), lambda g: (0, 0)),
            pl.BlockSpec((NC, r, 1), lambda g: (0, g, 0)),
        ],
        out_specs=pl.BlockSpec((NC, r, hh), lambda g: (0, g, 0)),
        out_shape=jax.ShapeDtypeStruct((NC, n_acc, hh), jnp.float32),
    )(x, W1, deg2)


def _tc_mid(acc, y1, deg2, W2, b1, *, n_acc):
    hh_in = y1.shape[2]
    h_out = W2.shape[1]
    hh_out = h_out // NC
    r = n_acc // 16

    def body(a_ref, y_ref, d_ref, w_ref, b_ref, o_ref):
        dinv = lax.rsqrt(d_ref[0] + d_ref[1] + 1.0)
        agg = jnp.concatenate([a_ref[0] + y_ref[0], a_ref[1] + y_ref[1]],
                              axis=1)
        h1 = jnp.maximum(dinv * agg + b_ref[...], 0.0)
        y2 = dinv * jnp.dot(h1, w_ref[...],
                            preferred_element_type=jnp.float32)
        o_ref[0] = y2[:, :hh_out]
        o_ref[1] = y2[:, hh_out:]

    return pl.pallas_call(
        body,
        grid=(n_acc // r,),
        in_specs=[
            pl.BlockSpec((NC, r, hh_in), lambda g: (0, g, 0)),
            pl.BlockSpec((NC, r, hh_in), lambda g: (0, g, 0)),
            pl.BlockSpec((NC, r, 1), lambda g: (0, g, 0)),
            pl.BlockSpec((NC * hh_in, h_out), lambda g: (0, 0)),
            pl.BlockSpec((1, NC * hh_in), lambda g: (0, 0)),
        ],
        out_specs=pl.BlockSpec((NC, r, hh_out), lambda g: (0, g, 0)),
        out_shape=jax.ShapeDtypeStruct((NC, n_acc, hh_out), jnp.float32),
    )(acc, y1, deg2, W2, b1)


def _tc_final(acc, y2, deg2, Wc, b2, bc, *, n, n_acc):
    hh_in = y2.shape[2]
    c_out = Wc.shape[1]
    r = n_acc // 16

    def body(a_ref, y_ref, d_ref, w_ref, b2_ref, bc_ref, o_ref):
        dinv = lax.rsqrt(d_ref[0] + d_ref[1] + 1.0)
        agg = jnp.concatenate([a_ref[0] + y_ref[0], a_ref[1] + y_ref[1]],
                              axis=1)
        h2 = jnp.maximum(dinv * agg + b2_ref[...], 0.0)
        logits = jnp.dot(h2, w_ref[...],
                         preferred_element_type=jnp.float32) + bc_ref[...]
        m = jnp.max(logits, axis=1, keepdims=True)
        lse = jnp.log(jnp.sum(jnp.exp(logits - m), axis=1, keepdims=True)) + m
        o_ref[...] = logits - lse

    return pl.pallas_call(
        body,
        grid=(n_acc // r,),
        in_specs=[
            pl.BlockSpec((NC, r, hh_in), lambda g: (0, g, 0)),
            pl.BlockSpec((NC, r, hh_in), lambda g: (0, g, 0)),
            pl.BlockSpec((NC, r, 1), lambda g: (0, g, 0)),
            pl.BlockSpec((NC * hh_in, c_out), lambda g: (0, 0)),
            pl.BlockSpec((1, NC * hh_in), lambda g: (0, 0)),
            pl.BlockSpec((1, c_out), lambda g: (0, 0)),
        ],
        out_specs=pl.BlockSpec((r, c_out), lambda g: (g, 0)),
        out_shape=jax.ShapeDtypeStruct((n, c_out), jnp.float32),
    )(acc, y2, deg2, Wc, b2, bc)


def kernel(x, edge_index, W1, b1, W2, b2, Wc, bc):
    n, _ = x.shape
    e = edge_index.shape[1]
    h1 = W1.shape[1]
    h2 = W2.shape[1]

    epb = NW * CHUNK * NBUF
    e_pad = ((e + epb - 1) // epb) * epb
    k = e_pad // (NW * CHUNK)
    k2 = e_pad // (NS * CHUNK)
    n_acc = ((n + 1 + 127) // 128) * 128

    padv = jnp.full((e_pad - e,), n, jnp.int32)
    src2d = jnp.concatenate([edge_index[0], padv]).reshape(e_pad // CHUNK, CHUNK)
    dst2d = jnp.concatenate([edge_index[1], padv]).reshape(e_pad // CHUNK, CHUNK)

    deg = _sc_degree(dst2d, n_acc=n_acc, k=k)
    deg2 = deg.reshape(NC, n_acc, 1)

    y1 = _tc_y1(x, W1, deg2, n_acc=n_acc)
    acc1 = _sc_gather_scatter(y1, src2d, dst2d, hh=h1 // NC, n_acc=n_acc,
                              k2=k2, nbuf=NBUF)
    y2 = _tc_mid(acc1, y1, deg2, W2, b1.reshape(1, -1), n_acc=n_acc)
    acc2 = _sc_gather_scatter(y2, src2d, dst2d, hh=h2 // NC, n_acc=n_acc,
                              k2=k2, nbuf=NBUF)
    return _tc_final(acc2, y2, deg2, Wc, b2.reshape(1, -1), bc.reshape(1, -1),
                     n=n, n_acc=n_acc)
